# baseline (speedup 1.0000x reference)
import functools

import jax
import jax.numpy as jnp
from jax import lax
from jax.experimental import pallas as pl
from jax.experimental.pallas import tpu as pltpu
from jax.experimental.pallas import tpu_sc as plsc

NE = 8
D = 1024
DFF = 2048
BLK = 128
NC = 2
NS = 16


def _cumsum0(a):
    n = a.shape[0]
    k = 1
    while k < n:
        shifted = jnp.concatenate(
            [jnp.zeros((k,) + a.shape[1:], a.dtype), a[:-k]], axis=0)
        a = a + shifted
        k *= 2
    return a


def _router_kernel(nblk, x_ref, wg_ref, rw_ref, topw_ref, rpos_ref, blk_ref):
    x = x_ref[...]
    logits = jnp.dot(x, wg_ref[...], preferred_element_type=jnp.float32)
    m = jnp.max(logits, axis=-1, keepdims=True)
    ex = jnp.exp(logits - m)
    rw = ex / jnp.sum(ex, axis=-1, keepdims=True)
    rw_ref[...] = rw

    lane = jax.lax.broadcasted_iota(jnp.int32, rw.shape, 1)
    i1 = jnp.argmax(rw, axis=-1)
    w1 = jnp.max(rw, axis=-1)
    oh1 = lane == i1[:, None]
    rw2 = jnp.where(oh1, -jnp.inf, rw)
    i2 = jnp.argmax(rw2, axis=-1)
    w2 = jnp.max(rw2, axis=-1)
    s = w1 + w2
    topw_ref[...] = jnp.stack([w1 / s, w2 / s], axis=0)

    a = oh1.astype(jnp.float32)
    b = (lane == i2[:, None]).astype(jnp.float32)
    ca = _cumsum0(a)
    cb = _cumsum0(b)
    counts = ca[-1:, :] + cb[-1:, :]
    nb = jnp.floor((counts + (BLK - 1)) * (1.0 / BLK)) * BLK
    tri = (jax.lax.broadcasted_iota(jnp.int32, (NE, NE), 0)
           < jax.lax.broadcasted_iota(jnp.int32, (NE, NE), 1)).astype(jnp.float32)
    pstart = jnp.dot(nb, tri, preferred_element_type=jnp.float32)
    excl = (ca - a) + (cb - b)
    p1 = jnp.sum(jnp.where(oh1, excl + pstart, 0.0), axis=-1)
    p2 = jnp.sum(jnp.where(lane == i2[:, None], excl + pstart, 0.0), axis=-1)
    rpos_ref[...] = jnp.stack([p1, p2], axis=0).astype(jnp.int32)

    total = jnp.sum(nb, axis=-1, keepdims=True)
    blkpos = (jax.lax.broadcasted_iota(jnp.int32, (1, nblk), 1)
              .astype(jnp.float32) * BLK)
    blkpos = jnp.minimum(blkpos, total - 1.0)
    acc = jnp.zeros((1, nblk), jnp.float32)
    for e in range(NE):
        acc = acc + (blkpos >= pstart[0:1, e:e + 1]).astype(jnp.float32)
    blk_ref[0:1, :nblk] = jnp.clip(acc - 1.0, 0.0, NE - 1).astype(jnp.int32)
    blk_ref[0:1, nblk:] = (total * (1.0 / BLK)).astype(jnp.int32)


def _sc_dispatch_body(tpw, x_hbm, rpos_hbm, xs_hbm, rows_v, idx_v, sem):
    wid = lax.axis_index("s") * NC + lax.axis_index("c")
    base = wid * tpw
    pltpu.sync_copy(x_hbm.at[pl.ds(base, tpw)], rows_v)
    pltpu.sync_copy(rpos_hbm.at[0, pl.ds(base, tpw)], idx_v)
    pltpu.async_copy(rows_v, xs_hbm.at[idx_v], sem).wait()
    pltpu.sync_copy(rpos_hbm.at[1, pl.ds(base, tpw)], idx_v)
    pltpu.async_copy(rows_v, xs_hbm.at[idx_v], sem).wait()


def _gemm_kernel(nblk, T, be_ref, rpos_ref, topw_ref,
                 xs_ref, w1_ref, w3_ref, w2_ref, y_ref, outs_s):
    f = pl.program_id(0)
    b = pl.program_id(1)

    @pl.when(b < be_ref[nblk])
    def _():
        xb = xs_ref[...]
        a = jnp.dot(xb, w1_ref[0], preferred_element_type=jnp.float32)
        c = jnp.dot(xb, w3_ref[0], preferred_element_type=jnp.float32)
        h = (a * jax.lax.logistic(a)) * c
        part = jnp.dot(h, w2_ref[0], preferred_element_type=jnp.float32)
        sl = pl.ds(b * BLK, BLK)

        @pl.when(f == 0)
        def _():
            outs_s[sl, :] = part

        @pl.when(f == 1)
        def _():
            outs_s[sl, :] = outs_s[sl, :] + part

    @pl.when(jnp.logical_and(f == 1, b == nblk - 1))
    def _():
        def tok(t, carry):
            r0 = rpos_ref[t]
            r1 = rpos_ref[T + t]
            y_ref[pl.ds(t, 1), :] = (
                outs_s[pl.ds(r0, 1), :] * topw_ref[t]
                + outs_s[pl.ds(r1, 1), :] * topw_ref[T + t])
            return carry

        jax.lax.fori_loop(0, T, tok, 0)


def kernel(x, Wg, W1, W3, W2):
    bsz, seq, d = x.shape
    xf = x.reshape(-1, d)
    T = xf.shape[0]
    nblk = (2 * T) // BLK + NE
    cap = nblk * BLK
    tpw = T // (NC * NS)

    rw, topw, rpos, blk = pl.pallas_call(
        functools.partial(_router_kernel, nblk),
        out_shape=(
            jax.ShapeDtypeStruct((T, NE), jnp.float32),
            jax.ShapeDtypeStruct((2, T), jnp.float32),
            jax.ShapeDtypeStruct((2, T), jnp.int32),
            jax.ShapeDtypeStruct((1, nblk + 1), jnp.int32),
        ),
    )(xf, Wg)

    mesh = plsc.VectorSubcoreMesh(core_axis_name="c", subcore_axis_name="s")

    xs = pl.kernel(
        functools.partial(_sc_dispatch_body, tpw),
        mesh=mesh,
        out_type=jax.ShapeDtypeStruct((cap, D), jnp.float32),
        scratch_types=[
            pltpu.VMEM((tpw, D), jnp.float32),
            pltpu.VMEM((tpw,), jnp.int32),
            pltpu.SemaphoreType.DMA,
        ],
    )(xf, rpos)

    hf = DFF // 2
    y = pl.pallas_call(
        functools.partial(_gemm_kernel, nblk, T),
        grid_spec=pltpu.PrefetchScalarGridSpec(
            num_scalar_prefetch=3,
            grid=(2, nblk),
            in_specs=[
                pl.BlockSpec((BLK, D), lambda f, b, be, rp, tw: (b, 0)),
                pl.BlockSpec((1, D, hf), lambda f, b, be, rp, tw: (be[b], 0, f)),
                pl.BlockSpec((1, D, hf), lambda f, b, be, rp, tw: (be[b], 0, f)),
                pl.BlockSpec((1, hf, D), lambda f, b, be, rp, tw: (be[b], f, 0)),
            ],
            out_specs=pl.BlockSpec((T, D), lambda f, b, be, rp, tw: (0, 0)),
            scratch_shapes=[pltpu.VMEM((cap, D), jnp.float32)],
        ),
        out_shape=jax.ShapeDtypeStruct((T, D), jnp.float32),
    )(blk.reshape(nblk + 1), rpos.reshape(2 * T), topw.reshape(2 * T),
      xs, W1, W3, W2)

    return y.reshape(bsz, seq, d), rw


# speedup vs baseline: 1.1125x; 1.1125x over previous
import functools

import jax
import jax.numpy as jnp
from jax import lax
from jax.experimental import pallas as pl
from jax.experimental.pallas import tpu as pltpu
from jax.experimental.pallas import tpu_sc as plsc

NE = 8
D = 1024
DFF = 2048
BLK = 128
NC = 2
NS = 16
SCL = 16


def _cumsum0(a):
    n = a.shape[0]
    k = 1
    while k < n:
        shifted = jnp.concatenate(
            [jnp.zeros((k,) + a.shape[1:], a.dtype), a[:-k]], axis=0)
        a = a + shifted
        k *= 2
    return a


def _router_kernel(nblk, x_ref, wg_ref, rw_ref, topw_ref, rpos_ref, blk_ref):
    x = x_ref[...]
    logits = jnp.dot(x, wg_ref[...], preferred_element_type=jnp.float32)
    m = jnp.max(logits, axis=-1, keepdims=True)
    ex = jnp.exp(logits - m)
    rw = ex / jnp.sum(ex, axis=-1, keepdims=True)
    rw_ref[...] = rw

    lane = jax.lax.broadcasted_iota(jnp.int32, rw.shape, 1)
    i1 = jnp.argmax(rw, axis=-1)
    w1 = jnp.max(rw, axis=-1)
    oh1 = lane == i1[:, None]
    rw2 = jnp.where(oh1, -jnp.inf, rw)
    i2 = jnp.argmax(rw2, axis=-1)
    w2 = jnp.max(rw2, axis=-1)
    s = w1 + w2
    topw_ref[...] = jnp.stack(
        [jnp.broadcast_to((w1 / s)[:, None], (x.shape[0], SCL)),
         jnp.broadcast_to((w2 / s)[:, None], (x.shape[0], SCL))], axis=0)

    a = oh1.astype(jnp.float32)
    b = (lane == i2[:, None]).astype(jnp.float32)
    ca = _cumsum0(a)
    cb = _cumsum0(b)
    counts = ca[-1:, :] + cb[-1:, :]
    nb = jnp.floor((counts + (BLK - 1)) * (1.0 / BLK)) * BLK
    tri = (jax.lax.broadcasted_iota(jnp.int32, (NE, NE), 0)
           < jax.lax.broadcasted_iota(jnp.int32, (NE, NE), 1)).astype(jnp.float32)
    pstart = jnp.dot(nb, tri, preferred_element_type=jnp.float32)
    excl = (ca - a) + (cb - b)
    p1 = jnp.sum(jnp.where(oh1, excl + pstart, 0.0), axis=-1)
    p2 = jnp.sum(jnp.where(lane == i2[:, None], excl + pstart, 0.0), axis=-1)
    rpos_ref[...] = jnp.stack([p1, p2], axis=0).astype(jnp.int32)

    total = jnp.sum(nb, axis=-1, keepdims=True)
    blkpos = (jax.lax.broadcasted_iota(jnp.int32, (1, nblk), 1)
              .astype(jnp.float32) * BLK)
    blkpos = jnp.minimum(blkpos, total - 1.0)
    acc = jnp.zeros((1, nblk), jnp.float32)
    for e in range(NE):
        acc = acc + (blkpos >= pstart[0:1, e:e + 1]).astype(jnp.float32)
    blk_ref[0:1, :nblk] = jnp.clip(acc - 1.0, 0.0, NE - 1).astype(jnp.int32)
    blk_ref[0:1, nblk:] = (total * (1.0 / BLK)).astype(jnp.int32)


def _sc_dispatch_body(tpw, x_hbm, rpos_hbm, xs_hbm, rows_v, idx_v, sem):
    wid = lax.axis_index("s") * NC + lax.axis_index("c")
    base = wid * tpw
    pltpu.sync_copy(x_hbm.at[pl.ds(base, tpw)], rows_v)
    pltpu.sync_copy(rpos_hbm.at[0, pl.ds(base, tpw)], idx_v)
    pltpu.async_copy(rows_v, xs_hbm.at[idx_v], sem).wait()
    pltpu.sync_copy(rpos_hbm.at[1, pl.ds(base, tpw)], idx_v)
    pltpu.async_copy(rows_v, xs_hbm.at[idx_v], sem).wait()


def _sc_collect_body(tpw, outs_hbm, rpos_hbm, w_hbm, y_hbm,
                     rows0_v, rows1_v, ybuf_v, idx_v, w0_v, w1_v, sem):
    wid = lax.axis_index("s") * NC + lax.axis_index("c")
    half = tpw // 2
    for h in range(2):
        hb = wid * tpw + h * half
        pltpu.sync_copy(rpos_hbm.at[0, pl.ds(hb, half)], idx_v)
        pltpu.async_copy(outs_hbm.at[idx_v], rows0_v, sem).wait()
        pltpu.sync_copy(rpos_hbm.at[1, pl.ds(hb, half)], idx_v)
        pltpu.async_copy(outs_hbm.at[idx_v], rows1_v, sem).wait()
        pltpu.sync_copy(w_hbm.at[0, pl.ds(hb, half)], w0_v)
        pltpu.sync_copy(w_hbm.at[1, pl.ds(hb, half)], w1_v)

        def tok_body(i, carry):
            w0 = w0_v[i, pl.ds(0, SCL)]
            w1 = w1_v[i, pl.ds(0, SCL)]
            for j in range(D // SCL):
                r0 = rows0_v[i, pl.ds(j * SCL, SCL)]
                r1 = rows1_v[i, pl.ds(j * SCL, SCL)]
                ybuf_v[i, pl.ds(j * SCL, SCL)] = r0 * w0 + r1 * w1
            return carry

        jax.lax.fori_loop(0, half, tok_body, 0)
        pltpu.sync_copy(ybuf_v, y_hbm.at[pl.ds(hb, half)])


def _gemm_kernel(nblk, be_ref, xs_ref, w1_ref, w3_ref, w2_ref, out_ref):
    @pl.when(pl.program_id(0) < be_ref[nblk])
    def _():
        xb = xs_ref[...]
        a = jnp.dot(xb, w1_ref[0], preferred_element_type=jnp.float32)
        c = jnp.dot(xb, w3_ref[0], preferred_element_type=jnp.float32)
        h = (a * jax.lax.logistic(a)) * c
        out_ref[...] = jnp.dot(h, w2_ref[0], preferred_element_type=jnp.float32)


def kernel(x, Wg, W1, W3, W2):
    bsz, seq, d = x.shape
    xf = x.reshape(-1, d)
    T = xf.shape[0]
    nblk = (2 * T) // BLK + NE
    cap = nblk * BLK
    tpw = T // (NC * NS)

    rw, topw, rpos, blk = pl.pallas_call(
        functools.partial(_router_kernel, nblk),
        out_shape=(
            jax.ShapeDtypeStruct((T, NE), jnp.float32),
            jax.ShapeDtypeStruct((2, T, SCL), jnp.float32),
            jax.ShapeDtypeStruct((2, T), jnp.int32),
            jax.ShapeDtypeStruct((1, nblk + 1), jnp.int32),
        ),
    )(xf, Wg)

    mesh = plsc.VectorSubcoreMesh(core_axis_name="c", subcore_axis_name="s")

    xs = pl.kernel(
        functools.partial(_sc_dispatch_body, tpw),
        mesh=mesh,
        out_type=jax.ShapeDtypeStruct((cap, D), jnp.float32),
        scratch_types=[
            pltpu.VMEM((tpw, D), jnp.float32),
            pltpu.VMEM((tpw,), jnp.int32),
            pltpu.SemaphoreType.DMA,
        ],
    )(xf, rpos)

    outs = pl.pallas_call(
        functools.partial(_gemm_kernel, nblk),
        grid_spec=pltpu.PrefetchScalarGridSpec(
            num_scalar_prefetch=1,
            grid=(nblk,),
            in_specs=[
                pl.BlockSpec((BLK, D), lambda b, be: (b, 0)),
                pl.BlockSpec((1, D, DFF), lambda b, be: (be[b], 0, 0)),
                pl.BlockSpec((1, D, DFF), lambda b, be: (be[b], 0, 0)),
                pl.BlockSpec((1, DFF, D), lambda b, be: (be[b], 0, 0)),
            ],
            out_specs=pl.BlockSpec((BLK, D), lambda b, be: (b, 0)),
        ),
        out_shape=jax.ShapeDtypeStruct((cap, D), jnp.float32),
    )(blk.reshape(nblk + 1), xs, W1, W3, W2)

    y = pl.kernel(
        functools.partial(_sc_collect_body, tpw),
        mesh=mesh,
        out_type=jax.ShapeDtypeStruct((T, D), jnp.float32),
        scratch_types=[
            pltpu.VMEM((tpw // 2, D), jnp.float32),
            pltpu.VMEM((tpw // 2, D), jnp.float32),
            pltpu.VMEM((tpw // 2, D), jnp.float32),
            pltpu.VMEM((tpw // 2,), jnp.int32),
            pltpu.VMEM((tpw // 2, SCL), jnp.float32),
            pltpu.VMEM((tpw // 2, SCL), jnp.float32),
            pltpu.SemaphoreType.DMA,
        ],
    )(outs, rpos, topw)

    return y.reshape(bsz, seq, d), rw
